# Initial kernel scaffold; baseline (speedup 1.0000x reference)
import jax
import jax.numpy as jnp
from jax.experimental import pallas as pl


def kernel(x, edge_index, batch, W1, b1, W2, b2, W3, b3, Wf1, bf1, Wf2, bf2):
    raise NotImplementedError("write your pallas kernel here")


# speedup vs baseline: 15.1750x; 15.1750x over previous
import jax
import jax.numpy as jnp
from jax import lax
from jax.experimental import pallas as pl
from jax.experimental.pallas import tpu as pltpu
from jax.experimental.pallas import tpu_sc as plsc

N = 10000
NPAD = 10240
RPT = NPAD // 16
D = 128
H = 64
G = 64
E = 320000
CH = 128
NTILES = 32
NCH = 79
EPAD = NTILES * NCH * CH
_HI = lax.Precision.HIGHEST


def _sc_mesh():
    return plsc.VectorSubcoreMesh(core_axis_name="c", subcore_axis_name="s")


def _deg_pass(dstp, zeros8, ones8):
    def body(dst_hbm, z_hbm, ones_hbm, out_hbm, dst_v, ones_v, acc):
        cid = lax.axis_index("c")
        sid = lax.axis_index("s")
        wid = cid * 16 + sid
        r0 = sid * RPT
        pltpu.sync_copy(z_hbm.at[pl.ds(r0, RPT)], acc.at[pl.ds(r0, RPT)])
        pltpu.sync_copy(ones_hbm, ones_v)
        pltpu.sync_copy(dst_hbm.at[wid], dst_v)
        plsc.subcore_barrier()

        def chunk(j, c):
            pltpu.sync_copy(ones_v, acc.at[dst_v.at[j]], add=True)
            return c

        lax.fori_loop(0, NCH, chunk, 0)
        plsc.subcore_barrier()
        pltpu.sync_copy(acc.at[pl.ds(r0, RPT)],
                        out_hbm.at[pl.ds(cid * NPAD + r0, RPT)])

    f = pl.kernel(
        body,
        out_type=jax.ShapeDtypeStruct((2 * NPAD, 8), jnp.float32),
        mesh=_sc_mesh(),
        compiler_params=pltpu.CompilerParams(use_tc_tiling_on_sc=False),
        scratch_types=[
            pltpu.VMEM((NCH, CH), jnp.int32),
            pltpu.VMEM((CH, 8), jnp.float32),
            pltpu.VMEM_SHARED((NPAD, 8), jnp.float32),
        ],
    )
    return f(dstp, zeros8, ones8)


def _agg_pass(y, zeros64, srcp, dstp):
    def body(y_hbm, z_hbm, src_hbm, dst_hbm, out_hbm, src_v, dst_v, buf, sem,
             acc):
        cid = lax.axis_index("c")
        sid = lax.axis_index("s")
        wid = cid * 16 + sid
        r0 = sid * RPT

        @pl.when(cid == 0)
        def _():
            pltpu.sync_copy(y_hbm.at[pl.ds(r0, RPT)], acc.at[pl.ds(r0, RPT)])

        @pl.when(cid == 1)
        def _():
            pltpu.sync_copy(z_hbm.at[pl.ds(r0, RPT)], acc.at[pl.ds(r0, RPT)])

        pltpu.sync_copy(src_hbm.at[wid], src_v)
        pltpu.sync_copy(dst_hbm.at[wid], dst_v)
        plsc.subcore_barrier()

        def chunk(j, c):
            pltpu.async_copy(y_hbm.at[src_v.at[j]], buf, sem).wait()
            pltpu.sync_copy(buf, acc.at[dst_v.at[j]], add=True)
            return c

        lax.fori_loop(0, NCH, chunk, 0)
        plsc.subcore_barrier()
        pltpu.sync_copy(acc.at[pl.ds(r0, RPT)],
                        out_hbm.at[pl.ds(cid * NPAD + r0, RPT)])

    f = pl.kernel(
        body,
        out_type=jax.ShapeDtypeStruct((2 * NPAD, H), jnp.float32),
        mesh=_sc_mesh(),
        compiler_params=pltpu.CompilerParams(use_tc_tiling_on_sc=False),
        scratch_types=[
            pltpu.VMEM((NCH, CH), jnp.int32),
            pltpu.VMEM((NCH, CH), jnp.int32),
            pltpu.VMEM((CH, H), jnp.float32),
            pltpu.SemaphoreType.DMA,
            pltpu.VMEM_SHARED((NPAD, H), jnp.float32),
        ],
    )
    return f(y, zeros64, srcp, dstp)


def _k1(xp, W1, deg0, deg1):
    def body(x_ref, w_ref, d0_ref, d1_ref, y_ref, dinv_ref):
        deg = d0_ref[:, 0:1] + d1_ref[:, 0:1] + 1.0
        rows = lax.broadcasted_iota(jnp.int32, (NPAD, 1), 0)
        dinv = jnp.where(rows < N, lax.rsqrt(deg), 0.0)
        xw = jnp.dot(x_ref[...], w_ref[...], precision=_HI,
                     preferred_element_type=jnp.float32)
        y_ref[...] = xw * dinv
        dinv_ref[...] = jnp.broadcast_to(dinv, (NPAD, 8))

    return pl.pallas_call(
        body,
        out_shape=(jax.ShapeDtypeStruct((NPAD, H), jnp.float32),
                   jax.ShapeDtypeStruct((NPAD, 8), jnp.float32)),
    )(xp, W1, deg0, deg1)


def _k23(a0, a1, dinv8, b, W):
    def body(a0_ref, a1_ref, dinv_ref, b_ref, w_ref, y_ref):
        dinv = dinv_ref[:, 0:1]
        h = jnp.maximum(dinv * (a0_ref[...] + a1_ref[...]) + b_ref[...], 0.0)
        y_ref[...] = dinv * jnp.dot(h, w_ref[...], precision=_HI,
                                    preferred_element_type=jnp.float32)

    return pl.pallas_call(
        body,
        out_shape=jax.ShapeDtypeStruct((NPAD, H), jnp.float32),
    )(a0, a1, dinv8, b, W)


def _k4(a0, a1, dinv8, b3, batchp, Wf1, bf1, Wf2, bf2):
    def body(a0_ref, a1_ref, dinv_ref, b_ref, batch_ref, wf1_ref, bf1_ref,
             wf2_ref, bf2_ref, out_ref):
        dinv = dinv_ref[:, 0:1]
        h = jnp.maximum(dinv * (a0_ref[...] + a1_ref[...]) + b_ref[...], 0.0)
        gids = lax.broadcasted_iota(jnp.int32, (G, NPAD), 0)
        mt = (gids == batch_ref[...]).astype(jnp.float32)
        sums = jnp.dot(mt, h, precision=_HI,
                       preferred_element_type=jnp.float32)
        counts = jnp.sum(mt, axis=1, keepdims=True)
        pooled = sums / jnp.maximum(counts, 1.0)
        hh = jnp.maximum(jnp.dot(pooled, wf1_ref[...], precision=_HI,
                                 preferred_element_type=jnp.float32)
                         + bf1_ref[...], 0.0)
        logits = jnp.dot(hh, wf2_ref[...], precision=_HI,
                         preferred_element_type=jnp.float32) + bf2_ref[...]
        ls = logits - jnp.max(logits, axis=1, keepdims=True)
        out_ref[...] = ls - jnp.log(jnp.sum(jnp.exp(ls), axis=1,
                                            keepdims=True))

    return pl.pallas_call(
        body,
        out_shape=jax.ShapeDtypeStruct((G, 2), jnp.float32),
    )(a0, a1, dinv8, b3, batchp, Wf1, bf1, Wf2, bf2)


def kernel(x, edge_index, batch, W1, b1, W2, b2, W3, b3, Wf1, bf1, Wf2, bf2):
    f32 = jnp.float32
    xp = jnp.pad(x, ((0, NPAD - N), (0, 0)))
    epad = jnp.full((EPAD - E,), N, jnp.int32)
    srcp = jnp.concatenate([edge_index[0], epad]).reshape(NTILES, NCH, CH)
    dstp = jnp.concatenate([edge_index[1], epad]).reshape(NTILES, NCH, CH)
    batchp = jnp.concatenate(
        [batch, jnp.full((NPAD - N,), -1, jnp.int32)]).reshape(1, NPAD)
    zeros64 = jnp.zeros((NPAD, H), f32)
    zeros8 = jnp.zeros((NPAD, 8), f32)
    ones8 = jnp.ones((CH, 8), f32)

    degp = _deg_pass(dstp, zeros8, ones8)
    y, dinv8 = _k1(xp, W1, degp[:NPAD], degp[NPAD:])
    a = _agg_pass(y, zeros64, srcp, dstp)
    y = _k23(a[:NPAD], a[NPAD:], dinv8, b1.reshape(1, H), W2)
    a = _agg_pass(y, zeros64, srcp, dstp)
    y = _k23(a[:NPAD], a[NPAD:], dinv8, b2.reshape(1, H), W3)
    a = _agg_pass(y, zeros64, srcp, dstp)
    return _k4(a[:NPAD], a[NPAD:], dinv8, b3.reshape(1, H), batchp,
               Wf1, bf1.reshape(1, 32), Wf2, bf2.reshape(1, 2))


# speedup vs baseline: 18.7870x; 1.2380x over previous
import jax
import jax.numpy as jnp
from jax import lax
from jax.experimental import pallas as pl
from jax.experimental.pallas import tpu as pltpu
from jax.experimental.pallas import tpu_sc as plsc

N = 10000
NPAD = 10240
RPT = NPAD // 16
D = 128
H = 64
G = 64
E = 320000
CH = 128
NTILES = 32
NCH = 79
EPAD = NTILES * NCH * CH
_HI = lax.Precision.HIGHEST


def _sc_mesh():
    return plsc.VectorSubcoreMesh(core_axis_name="c", subcore_axis_name="s")


def _deg_pass(dstp, zeros8, ones8):
    def body(dst_hbm, z_hbm, ones_hbm, out_hbm, dst_v, ones_v, acc):
        cid = lax.axis_index("c")
        sid = lax.axis_index("s")
        wid = cid * 16 + sid
        r0 = sid * RPT
        pltpu.sync_copy(z_hbm.at[pl.ds(r0, RPT)], acc.at[pl.ds(r0, RPT)])
        pltpu.sync_copy(ones_hbm, ones_v)
        pltpu.sync_copy(dst_hbm.at[wid], dst_v)
        plsc.subcore_barrier()

        def chunk(j, c):
            pltpu.sync_copy(ones_v, acc.at[dst_v.at[j]], add=True)
            return c

        lax.fori_loop(0, NCH, chunk, 0)
        plsc.subcore_barrier()
        pltpu.sync_copy(acc.at[pl.ds(r0, RPT)],
                        out_hbm.at[pl.ds(cid * NPAD + r0, RPT)])

    f = pl.kernel(
        body,
        out_type=jax.ShapeDtypeStruct((2 * NPAD, 8), jnp.float32),
        mesh=_sc_mesh(),
        compiler_params=pltpu.CompilerParams(use_tc_tiling_on_sc=False),
        scratch_types=[
            pltpu.VMEM((NCH, CH), jnp.int32),
            pltpu.VMEM((CH, 8), jnp.float32),
            pltpu.VMEM_SHARED((NPAD, 8), jnp.float32),
        ],
    )
    return f(dstp, zeros8, ones8)


def _agg_pass(y, zeros64, srcp, dstp):
    def body(y_hbm, z_hbm, src_hbm, dst_hbm, out_hbm, src_v, dst_v,
             buf0, buf1, s0, s1, acc):
        cid = lax.axis_index("c")
        sid = lax.axis_index("s")
        wid = cid * 16 + sid
        r0 = sid * RPT

        @pl.when(cid == 0)
        def _():
            pltpu.sync_copy(y_hbm.at[pl.ds(r0, RPT)], acc.at[pl.ds(r0, RPT)])

        @pl.when(cid == 1)
        def _():
            pltpu.sync_copy(z_hbm.at[pl.ds(r0, RPT)], acc.at[pl.ds(r0, RPT)])

        pltpu.sync_copy(src_hbm.at[wid], src_v)
        pltpu.sync_copy(dst_hbm.at[wid], dst_v)
        pltpu.async_copy(y_hbm.at[src_v.at[0]], buf0, s0)
        plsc.subcore_barrier()

        def chunk(j, c):
            even = (j % 2) == 0

            @pl.when(jnp.logical_and(j + 1 < NCH, even))
            def _():
                pltpu.async_copy(y_hbm.at[src_v.at[j + 1]], buf1, s1)

            @pl.when(jnp.logical_and(j + 1 < NCH, jnp.logical_not(even)))
            def _():
                pltpu.async_copy(y_hbm.at[src_v.at[j + 1]], buf0, s0)

            @pl.when(even)
            def _():
                pltpu.make_async_copy(y_hbm.at[src_v.at[j]], buf0, s0).wait()
                pltpu.sync_copy(buf0, acc.at[dst_v.at[j]], add=True)

            @pl.when(jnp.logical_not(even))
            def _():
                pltpu.make_async_copy(y_hbm.at[src_v.at[j]], buf1, s1).wait()
                pltpu.sync_copy(buf1, acc.at[dst_v.at[j]], add=True)

            return c

        lax.fori_loop(0, NCH, chunk, 0)
        plsc.subcore_barrier()
        pltpu.sync_copy(acc.at[pl.ds(r0, RPT)],
                        out_hbm.at[pl.ds(cid * NPAD + r0, RPT)])

    f = pl.kernel(
        body,
        out_type=jax.ShapeDtypeStruct((2 * NPAD, H), jnp.float32),
        mesh=_sc_mesh(),
        compiler_params=pltpu.CompilerParams(use_tc_tiling_on_sc=False),
        scratch_types=[
            pltpu.VMEM((NCH, CH), jnp.int32),
            pltpu.VMEM((NCH, CH), jnp.int32),
            pltpu.VMEM((CH, H), jnp.float32),
            pltpu.VMEM((CH, H), jnp.float32),
            pltpu.SemaphoreType.DMA,
            pltpu.SemaphoreType.DMA,
            pltpu.VMEM_SHARED((NPAD, H), jnp.float32),
        ],
    )
    return f(y, zeros64, srcp, dstp)


def _k1(xp, W1, deg0, deg1):
    def body(x_ref, w_ref, d0_ref, d1_ref, y_ref, dinv_ref):
        deg = d0_ref[:, 0:1] + d1_ref[:, 0:1] + 1.0
        rows = lax.broadcasted_iota(jnp.int32, (NPAD, 1), 0)
        dinv = jnp.where(rows < N, lax.rsqrt(deg), 0.0)
        xw = jnp.dot(x_ref[...], w_ref[...], precision=_HI,
                     preferred_element_type=jnp.float32)
        y_ref[...] = xw * dinv
        dinv_ref[...] = jnp.broadcast_to(dinv, (NPAD, 8))

    return pl.pallas_call(
        body,
        out_shape=(jax.ShapeDtypeStruct((NPAD, H), jnp.float32),
                   jax.ShapeDtypeStruct((NPAD, 8), jnp.float32)),
    )(xp, W1, deg0, deg1)


def _k23(a0, a1, dinv8, b, W):
    def body(a0_ref, a1_ref, dinv_ref, b_ref, w_ref, y_ref):
        dinv = dinv_ref[:, 0:1]
        h = jnp.maximum(dinv * (a0_ref[...] + a1_ref[...]) + b_ref[...], 0.0)
        y_ref[...] = dinv * jnp.dot(h, w_ref[...], precision=_HI,
                                    preferred_element_type=jnp.float32)

    return pl.pallas_call(
        body,
        out_shape=jax.ShapeDtypeStruct((NPAD, H), jnp.float32),
    )(a0, a1, dinv8, b, W)


def _k4(a0, a1, dinv8, b3, batchp, Wf1, bf1, Wf2, bf2):
    def body(a0_ref, a1_ref, dinv_ref, b_ref, batch_ref, wf1_ref, bf1_ref,
             wf2_ref, bf2_ref, out_ref):
        dinv = dinv_ref[:, 0:1]
        h = jnp.maximum(dinv * (a0_ref[...] + a1_ref[...]) + b_ref[...], 0.0)
        gids = lax.broadcasted_iota(jnp.int32, (G, NPAD), 0)
        mt = (gids == batch_ref[...]).astype(jnp.float32)
        sums = jnp.dot(mt, h, precision=_HI,
                       preferred_element_type=jnp.float32)
        counts = jnp.sum(mt, axis=1, keepdims=True)
        pooled = sums / jnp.maximum(counts, 1.0)
        hh = jnp.maximum(jnp.dot(pooled, wf1_ref[...], precision=_HI,
                                 preferred_element_type=jnp.float32)
                         + bf1_ref[...], 0.0)
        logits = jnp.dot(hh, wf2_ref[...], precision=_HI,
                         preferred_element_type=jnp.float32) + bf2_ref[...]
        ls = logits - jnp.max(logits, axis=1, keepdims=True)
        out_ref[...] = ls - jnp.log(jnp.sum(jnp.exp(ls), axis=1,
                                            keepdims=True))

    return pl.pallas_call(
        body,
        out_shape=jax.ShapeDtypeStruct((G, 2), jnp.float32),
    )(a0, a1, dinv8, b3, batchp, Wf1, bf1, Wf2, bf2)


def kernel(x, edge_index, batch, W1, b1, W2, b2, W3, b3, Wf1, bf1, Wf2, bf2):
    f32 = jnp.float32
    xp = jnp.pad(x, ((0, NPAD - N), (0, 0)))
    epad = jnp.full((EPAD - E,), N, jnp.int32)
    srcp = jnp.concatenate([edge_index[0], epad]).reshape(NTILES, NCH, CH)
    dstp = jnp.concatenate([edge_index[1], epad]).reshape(NTILES, NCH, CH)
    batchp = jnp.concatenate(
        [batch, jnp.full((NPAD - N,), -1, jnp.int32)]).reshape(1, NPAD)
    zeros64 = jnp.zeros((NPAD, H), f32)
    zeros8 = jnp.zeros((NPAD, 8), f32)
    ones8 = jnp.ones((CH, 8), f32)

    degp = _deg_pass(dstp, zeros8, ones8)
    y, dinv8 = _k1(xp, W1, degp[:NPAD], degp[NPAD:])
    a = _agg_pass(y, zeros64, srcp, dstp)
    y = _k23(a[:NPAD], a[NPAD:], dinv8, b1.reshape(1, H), W2)
    a = _agg_pass(y, zeros64, srcp, dstp)
    y = _k23(a[:NPAD], a[NPAD:], dinv8, b2.reshape(1, H), W3)
    a = _agg_pass(y, zeros64, srcp, dstp)
    return _k4(a[:NPAD], a[NPAD:], dinv8, b3.reshape(1, H), batchp,
               Wf1, bf1.reshape(1, 32), Wf2, bf2.reshape(1, 2))
